# baseline (speedup 1.0000x reference)
import jax
import jax.numpy as jnp
from jax import lax
from jax.experimental import pallas as pl
from jax.experimental.pallas import tpu as pltpu
from jax.experimental.pallas import tpu_sc as plsc

BSZ = 4
SEQ = 4096
D_MODEL = 1024
NC = 2
NS = 16
NW = NC * NS
ROWS_PER_W = SEQ // NW
CHUNK = 32
NBUF = 3
NCHUNK = ROWS_PER_W // CHUNK


def _pe_broadcast_body(pe_hbm, out_hbm, buf0, buf1, buf2, sem_r, sem_w):
    bufs = (buf0, buf1, buf2)
    wid = lax.axis_index("s") * NC + lax.axis_index("c")
    base = wid * ROWS_PER_W
    reads = [None] * NCHUNK
    writes = [[] for _ in range(NCHUNK)]
    for i in range(NBUF):
        reads[i] = pltpu.async_copy(
            pe_hbm.at[pl.ds(base + i * CHUNK, CHUNK)], bufs[i], sem_r)
    for i in range(NCHUNK):
        buf = bufs[i % NBUF]
        reads[i].wait()
        for b in range(BSZ):
            writes[i].append(pltpu.async_copy(
                buf, out_hbm.at[b, pl.ds(base + i * CHUNK, CHUNK)], sem_w))
        nxt = i + NBUF
        if nxt < NCHUNK:
            for w in writes[i]:
                w.wait()
            reads[nxt] = pltpu.async_copy(
                pe_hbm.at[pl.ds(base + nxt * CHUNK, CHUNK)],
                bufs[nxt % NBUF], sem_r)
    for i in range(max(NCHUNK - NBUF, 0), NCHUNK):
        for w in writes[i]:
            w.wait()


@jax.jit
def _pe_broadcast(pe):
    mesh = plsc.VectorSubcoreMesh(core_axis_name="c", subcore_axis_name="s",
                                  num_cores=NC, num_subcores=NS)
    f = pl.kernel(
        _pe_broadcast_body,
        mesh=mesh,
        out_type=jax.ShapeDtypeStruct((BSZ, SEQ, D_MODEL), jnp.float32),
        scratch_types=[
            pltpu.VMEM((CHUNK, D_MODEL), jnp.float32),
            pltpu.VMEM((CHUNK, D_MODEL), jnp.float32),
            pltpu.VMEM((CHUNK, D_MODEL), jnp.float32),
            pltpu.SemaphoreType.DMA,
            pltpu.SemaphoreType.DMA,
        ],
    )
    return f(pe)


def kernel(input, pe):
    del input
    return _pe_broadcast(pe)


# speedup vs baseline: 1.0115x; 1.0115x over previous
import jax
import jax.numpy as jnp
from jax import lax
from jax.experimental import pallas as pl
from jax.experimental.pallas import tpu as pltpu
from jax.experimental.pallas import tpu_sc as plsc

BSZ = 4
SEQ = 4096
D_MODEL = 1024
NC = 2
NS = 16
NW = NC * NS
ROWS_PER_W = SEQ // NW
CHUNK = 64


def _pe_broadcast_body(pe_hbm, out_hbm, buf0, buf1, sem_r0, sem_r1, sem_w):
    wid = lax.axis_index("s") * NC + lax.axis_index("c")
    base = wid * ROWS_PER_W
    r0 = pltpu.async_copy(pe_hbm.at[pl.ds(base, CHUNK)], buf0, sem_r0)
    r1 = pltpu.async_copy(pe_hbm.at[pl.ds(base + CHUNK, CHUNK)], buf1, sem_r1)
    writes = []
    r0.wait()
    for b in range(BSZ):
        writes.append(pltpu.async_copy(buf0, out_hbm.at[b, pl.ds(base, CHUNK)], sem_w))
    r1.wait()
    for b in range(BSZ):
        writes.append(pltpu.async_copy(buf1, out_hbm.at[b, pl.ds(base + CHUNK, CHUNK)], sem_w))
    for w in writes:
        w.wait()


@jax.jit
def _pe_broadcast(pe):
    mesh = plsc.VectorSubcoreMesh(core_axis_name="c", subcore_axis_name="s",
                                  num_cores=NC, num_subcores=NS)
    f = pl.kernel(
        _pe_broadcast_body,
        mesh=mesh,
        out_type=jax.ShapeDtypeStruct((BSZ, SEQ, D_MODEL), jnp.float32),
        scratch_types=[
            pltpu.VMEM((CHUNK, D_MODEL), jnp.float32),
            pltpu.VMEM((CHUNK, D_MODEL), jnp.float32),
            pltpu.SemaphoreType.DMA,
            pltpu.SemaphoreType.DMA,
            pltpu.SemaphoreType.DMA,
        ],
    )
    return f(pe)


def kernel(input, pe):
    del input
    return _pe_broadcast(pe)
